# baseline (speedup 1.0000x reference)
import functools

import jax
import jax.numpy as jnp
from jax import lax
from jax.experimental import pallas as pl
from jax.experimental.pallas import tpu as pltpu
from jax.experimental.pallas import tpu_sc as plsc

_N = 10000
_E = 320000
_D = 128
_NC = 2
_NS = 16
_NW = _NC * _NS
_CH = 128
_RT = 320
_RTD = _RT + 8
_NPAD = _NW * _RT
_DUMP = _N

_CHB = 4096
_NCHB = 79
_EPAD = _NCHB * _CHB
_FL = 2048
_STGN = 3 * _FL + 160

_mesh = plsc.VectorSubcoreMesh(core_axis_name="c", subcore_axis_name="s")
_params = pltpu.CompilerParams(needs_layout_passes=False)

_LANES = tuple(range(16))


@functools.partial(
    pl.kernel,
    mesh=_mesh,
    compiler_params=_params,
    out_type=(
        jax.ShapeDtypeStruct((_NW * _EPAD,), jnp.int32),
        jax.ShapeDtypeStruct((_NW * _EPAD,), jnp.int32),
        jax.ShapeDtypeStruct((_NW * 16,), jnp.int32),
        jax.ShapeDtypeStruct((_NPAD, 16), jnp.float32),
    ),
    scratch_types=[
        pltpu.VMEM((_CHB,), jnp.int32),
        pltpu.VMEM((_CHB,), jnp.int32),
        pltpu.VMEM((_STGN,), jnp.int32),
        pltpu.VMEM((_STGN,), jnp.int32),
        pltpu.VMEM((_RTD, 16), jnp.float32),
        pltpu.VMEM((16,), jnp.int32),
        pltpu.SemaphoreType.DMA,
        pltpu.SemaphoreType.DMA,
    ],
)
def _bin_call(src_hbm, dst_hbm, bsrc_hbm, bdst_hbm, cnt_hbm, deg_hbm,
              srcv, dstv, ssrc, sdst, degv, cstg, sem1, sem2):
    c = lax.axis_index("c")
    s = lax.axis_index("s")
    w = c * _NS + s
    lo = w * _RT

    def zdeg(i, _):
        degv[i, :] = jnp.zeros((16,), jnp.float32)
        return 0

    lax.fori_loop(0, _RTD, zdeg, 0)

    ones = jnp.ones((16,), jnp.float32)
    lanes = lax.iota(jnp.int32, 16)

    def chunk(j, carry):
        curv, fb = carry
        base = pl.multiple_of(j * _CHB, _CHB)
        d1 = pltpu.async_copy(src_hbm.at[pl.ds(base, _CHB)], srcv, sem1)
        d2 = pltpu.async_copy(dst_hbm.at[pl.ds(base, _CHB)], dstv, sem2)
        d1.wait()
        d2.wait()

        run = jnp.zeros((16,), jnp.int32)
        for k in range(_CHB // 16):
            dv = dstv[pl.ds(k * 16, 16)]
            sv = srcv[pl.ds(k * 16, 16)]
            own = (dv >= lo) & (dv < lo + _RT)
            dl = jnp.where(own, dv - lo, _RT)
            plsc.addupdate_scatter(degv, [dl, lanes], ones, mask=own)
            cs = plsc.cumsum(jnp.where(own, jnp.int32(1), jnp.int32(0)))
            pos = curv + run + cs - 1
            plsc.store_scatter(ssrc, [pos], sv, mask=own)
            plsc.store_scatter(sdst, [pos], dl, mask=own)
            run = run + plsc.all_reduce_population_count(own)

        curv = curv + run
        for _rep in range(2):
            flush = jnp.any(curv >= _FL)

            @pl.when(flush)
            def _():
                fbo = pl.multiple_of(w * _EPAD + fb, _CH)
                pltpu.sync_copy(ssrc.at[pl.ds(0, _FL)],
                                bsrc_hbm.at[pl.ds(fbo, _FL)])
                pltpu.sync_copy(sdst.at[pl.ds(0, _FL)],
                                bdst_hbm.at[pl.ds(fbo, _FL)])
                for t in range(2 * (_FL // 16) + 10):
                    tsl = pl.ds(t * 16, 16)
                    ssl = pl.ds(_FL + t * 16, 16)
                    vs = ssrc[ssl]
                    vd = sdst[ssl]
                    ssrc[tsl] = vs
                    sdst[tsl] = vd

            fb = jnp.where(flush, fb + _FL, fb)
            curv = jnp.where(flush, curv - _FL, curv)
        return curv, fb

    curv, fb = lax.fori_loop(
        0, _NCHB, chunk, (jnp.zeros((16,), jnp.int32), jnp.int32(0)))

    zsrc = jnp.zeros((16,), jnp.int32)
    zdst = jnp.full((16,), _RT, jnp.int32)
    for k in range(_CH // 16):
        tpos = lanes + curv + k * 16
        plsc.store_scatter(ssrc, [tpos], zsrc)
        plsc.store_scatter(sdst, [tpos], zdst)

    for t in range(_FL // _CH):
        @pl.when(jnp.any(curv > t * _CH))
        def _():
            fbo = pl.multiple_of(w * _EPAD + fb + t * _CH, _CH)
            pltpu.sync_copy(ssrc.at[pl.ds(t * _CH, _CH)],
                            bsrc_hbm.at[pl.ds(fbo, _CH)])
            pltpu.sync_copy(sdst.at[pl.ds(t * _CH, _CH)],
                            bdst_hbm.at[pl.ds(fbo, _CH)])

    cstg[...] = curv + fb
    pltpu.sync_copy(cstg, cnt_hbm.at[pl.ds(pl.multiple_of(w * 16, 16), 16)])
    pltpu.sync_copy(degv.at[pl.ds(0, _RT)],
                    deg_hbm.at[pl.ds(pl.multiple_of(lo, _RT), _RT)])


_MAXCH = _EPAD // _CH


@functools.partial(
    pl.kernel,
    mesh=_mesh,
    compiler_params=_params,
    out_type=jax.ShapeDtypeStruct((_NPAD, _D), jnp.float32),
    scratch_types=[
        pltpu.VMEM((_CH,), jnp.int32),
        pltpu.VMEM((_CH,), jnp.int32),
        pltpu.VMEM((16,), jnp.int32),
        pltpu.VMEM((_CH, _D), jnp.float32),
        pltpu.VMEM((_RTD, _D), jnp.float32),
        pltpu.SemaphoreType.DMA,
        pltpu.SemaphoreType.DMA,
    ],
)
def _prop_call(g_hbm, bsrc_hbm, bdst_hbm, cnt_hbm, out_hbm,
               srcv, dstv, cntv, rows, acc, sem, sem2):
    c = lax.axis_index("c")
    s = lax.axis_index("s")
    w = c * _NS + s

    def zacc(i, _):
        for f in range(_D // 16):
            acc[i, pl.ds(f * 16, 16)] = jnp.zeros((16,), jnp.float32)
        return 0

    lax.fori_loop(0, _RTD, zacc, 0)

    pltpu.sync_copy(cnt_hbm.at[pl.ds(pl.multiple_of(w * 16, 16), 16)], cntv)
    cnt = cntv[...]

    lanes = lax.iota(jnp.int32, 16)
    rots = [lax.rem(lanes + r, 16) for r in range(16)]

    _BLK = 8

    def block(b, _):
        @pl.when(jnp.any(cnt > b * (_BLK * _CH)))
        def _():
            def chunk(j, _):
                @pl.when(jnp.any(cnt > j * _CH))
                def _():
                    base = pl.multiple_of(w * _EPAD + j * _CH, _CH)
                    d1 = pltpu.async_copy(
                        bsrc_hbm.at[pl.ds(base, _CH)], srcv, sem)
                    d2 = pltpu.async_copy(
                        bdst_hbm.at[pl.ds(base, _CH)], dstv, sem2)
                    d1.wait()
                    d2.wait()
                    pltpu.async_copy(g_hbm.at[srcv], rows, sem).wait()

                    for g in range(_CH // 16):
                        dlv = dstv[pl.ds(g * 16, 16)]
                        ev = lanes + (g * 16)

                        def fgroup(f, _):
                            fb16 = f * 16
                            for r in range(16):
                                feat = fb16 + rots[r]
                                vals = plsc.load_gather(rows, [ev, feat])
                                plsc.addupdate_scatter(acc, [dlv, feat], vals)
                            return 0

                        lax.fori_loop(0, _D // 16, fgroup, 0)
                return 0

            lax.fori_loop(b * _BLK, (b + 1) * _BLK, chunk, 0)
        return 0

    lax.fori_loop(0, _MAXCH // _BLK, block, 0)

    pltpu.sync_copy(acc.at[pl.ds(0, _RT)],
                    out_hbm.at[pl.ds(pl.multiple_of(w * _RT, _RT), _RT)])


def _norm_from(dp_ref):
    deg = jnp.sum(dp_ref[...], axis=1, keepdims=True)
    return lax.rsqrt(jnp.maximum(deg, 1.0))


def _pre_body(dp_ref, x_ref, w1_ref, g1_ref):
    norm = _norm_from(dp_ref)
    g1_ref[...] = lax.dot_general(
        x_ref[...] * norm, w1_ref[...], (((1,), (0,)), ((), ())),
        precision=lax.Precision.HIGHEST, preferred_element_type=jnp.float32)


def _mid_body(p_ref, dp_ref, b1_ref, w2_ref, g2_ref):
    norm = _norm_from(dp_ref)
    h = jnp.maximum(p_ref[...] * norm + b1_ref[...], 0.0)
    g2_ref[...] = lax.dot_general(
        h * norm, w2_ref[...], (((1,), (0,)), ((), ())),
        precision=lax.Precision.HIGHEST, preferred_element_type=jnp.float32)


def _post_body(q_ref, dp_ref, b2_ref, out_ref):
    norm = _norm_from(dp_ref)
    out_ref[...] = q_ref[...] * norm + b2_ref[...]


_BR = 1000


def _row_spec(cols):
    return pl.BlockSpec((_BR, cols), lambda i: (i, 0))


def _full_spec(r, cols):
    return pl.BlockSpec((r, cols), lambda i: (0, 0))


_pre_call = pl.pallas_call(
    _pre_body,
    grid=(_N // _BR,),
    in_specs=[_row_spec(16), _row_spec(_D), _full_spec(_D, _D)],
    out_specs=_row_spec(_D),
    out_shape=jax.ShapeDtypeStruct((_N, _D), jnp.float32),
)

_mid_call = pl.pallas_call(
    _mid_body,
    grid=(_N // _BR,),
    in_specs=[_row_spec(_D), _row_spec(16), _full_spec(1, _D),
              _full_spec(_D, _D)],
    out_specs=_row_spec(_D),
    out_shape=jax.ShapeDtypeStruct((_N, _D), jnp.float32),
)

_post_call = pl.pallas_call(
    _post_body,
    grid=(_N // _BR,),
    in_specs=[_row_spec(_D), _row_spec(16), _full_spec(1, _D)],
    out_specs=_row_spec(_D),
    out_shape=jax.ShapeDtypeStruct((_N, _D), jnp.float32),
)


@jax.jit
def kernel(features, edge_index, W1, b1, W2, b2):
    src = edge_index[0]
    dst = edge_index[1]
    pad = _EPAD - _E
    src_p = jnp.concatenate([src, jnp.zeros((pad,), jnp.int32)])
    dst_p = jnp.concatenate([dst, jnp.full((pad,), _DUMP, jnp.int32)])

    bsrc, bdst, cnt, deg = _bin_call(src_p, dst_p)
    dp = deg[:_N]

    g1 = _pre_call(dp, features, W1)
    p = _prop_call(g1, bsrc, bdst, cnt)
    g2 = _mid_call(p[:_N], dp, b1.reshape(1, _D), W2)
    q = _prop_call(g2, bsrc, bdst, cnt)
    out = _post_call(q[:_N], dp, b2.reshape(1, _D))
    return out


# speedup vs baseline: 1.0106x; 1.0106x over previous
import functools

import jax
import jax.numpy as jnp
from jax import lax
from jax.experimental import pallas as pl
from jax.experimental.pallas import tpu as pltpu
from jax.experimental.pallas import tpu_sc as plsc

_N = 10000
_E = 320000
_D = 128
_NC = 2
_NS = 16
_NW = _NC * _NS
_CH = 128
_RT = 320
_RTD = _RT + 8
_NPAD = _NW * _RT
_DUMP = _N

_CHB = 2048
_NCHB = 158
_EPAD = _NCHB * _CHB
_FL = 2048
_STGN = 2 * _FL + 160

_mesh = plsc.VectorSubcoreMesh(core_axis_name="c", subcore_axis_name="s")
_params = pltpu.CompilerParams(needs_layout_passes=False)

_LANES = tuple(range(16))


@functools.partial(
    pl.kernel,
    mesh=_mesh,
    compiler_params=_params,
    out_type=(
        jax.ShapeDtypeStruct((_NW * _EPAD,), jnp.int32),
        jax.ShapeDtypeStruct((_NW * _EPAD,), jnp.int32),
        jax.ShapeDtypeStruct((_NW * 16,), jnp.int32),
        jax.ShapeDtypeStruct((_NPAD, 16), jnp.float32),
    ),
    scratch_types=[
        pltpu.VMEM((_CHB,), jnp.int32),
        pltpu.VMEM((_CHB,), jnp.int32),
        pltpu.VMEM((_STGN,), jnp.int32),
        pltpu.VMEM((_STGN,), jnp.int32),
        pltpu.VMEM((_RTD, 16), jnp.float32),
        pltpu.VMEM((16,), jnp.int32),
        pltpu.SemaphoreType.DMA,
        pltpu.SemaphoreType.DMA,
    ],
)
def _bin_call(src_hbm, dst_hbm, bsrc_hbm, bdst_hbm, cnt_hbm, deg_hbm,
              srcv, dstv, ssrc, sdst, degv, cstg, sem1, sem2):
    c = lax.axis_index("c")
    s = lax.axis_index("s")
    w = c * _NS + s
    lo = w * _RT

    def zdeg(i, _):
        degv[i, :] = jnp.zeros((16,), jnp.float32)
        return 0

    lax.fori_loop(0, _RTD, zdeg, 0)

    ones = jnp.ones((16,), jnp.float32)
    lanes = lax.iota(jnp.int32, 16)

    def chunk(j, carry):
        curv, fb = carry
        base = pl.multiple_of(j * _CHB, _CHB)
        d1 = pltpu.async_copy(src_hbm.at[pl.ds(base, _CHB)], srcv, sem1)
        d2 = pltpu.async_copy(dst_hbm.at[pl.ds(base, _CHB)], dstv, sem2)
        d1.wait()
        d2.wait()

        run = jnp.zeros((16,), jnp.int32)
        for k in range(_CHB // 16):
            dv = dstv[pl.ds(k * 16, 16)]
            sv = srcv[pl.ds(k * 16, 16)]
            own = (dv >= lo) & (dv < lo + _RT)
            dl = jnp.where(own, dv - lo, _RT)
            plsc.addupdate_scatter(degv, [dl, lanes], ones, mask=own)
            cs = plsc.cumsum(jnp.where(own, jnp.int32(1), jnp.int32(0)))
            pos = curv + run + cs - 1
            plsc.store_scatter(ssrc, [pos], sv, mask=own)
            plsc.store_scatter(sdst, [pos], dl, mask=own)
            run = run + plsc.all_reduce_population_count(own)

        curv = curv + run
        flush = jnp.any(curv >= _FL)

        @pl.when(flush)
        def _():
            fbo = pl.multiple_of(w * _EPAD + fb, _CH)
            pltpu.sync_copy(ssrc.at[pl.ds(0, _FL)],
                            bsrc_hbm.at[pl.ds(fbo, _FL)])
            pltpu.sync_copy(sdst.at[pl.ds(0, _FL)],
                            bdst_hbm.at[pl.ds(fbo, _FL)])
            for t in range(_FL // 16 + 10):
                tsl = pl.ds(t * 16, 16)
                ssl = pl.ds(_FL + t * 16, 16)
                vs = ssrc[ssl]
                vd = sdst[ssl]
                ssrc[tsl] = vs
                sdst[tsl] = vd

        fb = jnp.where(flush, fb + _FL, fb)
        curv = jnp.where(flush, curv - _FL, curv)
        return curv, fb

    curv, fb = lax.fori_loop(
        0, _NCHB, chunk, (jnp.zeros((16,), jnp.int32), jnp.int32(0)))

    zsrc = jnp.zeros((16,), jnp.int32)
    zdst = jnp.full((16,), _RT, jnp.int32)
    for k in range(_CH // 16):
        tpos = lanes + curv + k * 16
        plsc.store_scatter(ssrc, [tpos], zsrc)
        plsc.store_scatter(sdst, [tpos], zdst)

    for t in range(_FL // _CH):
        @pl.when(jnp.any(curv > t * _CH))
        def _():
            fbo = pl.multiple_of(w * _EPAD + fb + t * _CH, _CH)
            pltpu.sync_copy(ssrc.at[pl.ds(t * _CH, _CH)],
                            bsrc_hbm.at[pl.ds(fbo, _CH)])
            pltpu.sync_copy(sdst.at[pl.ds(t * _CH, _CH)],
                            bdst_hbm.at[pl.ds(fbo, _CH)])

    cstg[...] = curv + fb
    pltpu.sync_copy(cstg, cnt_hbm.at[pl.ds(pl.multiple_of(w * 16, 16), 16)])
    pltpu.sync_copy(degv.at[pl.ds(0, _RT)],
                    deg_hbm.at[pl.ds(pl.multiple_of(lo, _RT), _RT)])


_MAXCH = _EPAD // _CH


@functools.partial(
    pl.kernel,
    mesh=_mesh,
    compiler_params=_params,
    out_type=jax.ShapeDtypeStruct((_NPAD, _D), jnp.float32),
    scratch_types=[
        pltpu.VMEM((_CH,), jnp.int32),
        pltpu.VMEM((_CH,), jnp.int32),
        pltpu.VMEM((16,), jnp.int32),
        pltpu.VMEM((_CH, _D), jnp.float32),
        pltpu.VMEM((_RTD, _D), jnp.float32),
        pltpu.SemaphoreType.DMA,
        pltpu.SemaphoreType.DMA,
    ],
)
def _prop_call(g_hbm, bsrc_hbm, bdst_hbm, cnt_hbm, out_hbm,
               srcv, dstv, cntv, rows, acc, sem, sem2):
    c = lax.axis_index("c")
    s = lax.axis_index("s")
    w = c * _NS + s

    def zacc(i, _):
        for f in range(_D // 16):
            acc[i, pl.ds(f * 16, 16)] = jnp.zeros((16,), jnp.float32)
        return 0

    lax.fori_loop(0, _RTD, zacc, 0)

    pltpu.sync_copy(cnt_hbm.at[pl.ds(pl.multiple_of(w * 16, 16), 16)], cntv)
    cnt = cntv[...]

    lanes = lax.iota(jnp.int32, 16)
    rots = [lax.rem(lanes + r, 16) for r in range(16)]

    _BLK = 8

    def block(b, _):
        @pl.when(jnp.any(cnt > b * (_BLK * _CH)))
        def _():
            def chunk(j, _):
                @pl.when(jnp.any(cnt > j * _CH))
                def _():
                    base = pl.multiple_of(w * _EPAD + j * _CH, _CH)
                    d1 = pltpu.async_copy(
                        bsrc_hbm.at[pl.ds(base, _CH)], srcv, sem)
                    d2 = pltpu.async_copy(
                        bdst_hbm.at[pl.ds(base, _CH)], dstv, sem2)
                    d1.wait()
                    d2.wait()
                    pltpu.async_copy(g_hbm.at[srcv], rows, sem).wait()

                    for g in range(_CH // 16):
                        dlv = dstv[pl.ds(g * 16, 16)]
                        ev = lanes + (g * 16)

                        def fgroup(f, _):
                            fb16 = f * 16
                            for r in range(16):
                                feat = fb16 + rots[r]
                                vals = plsc.load_gather(rows, [ev, feat])
                                plsc.addupdate_scatter(acc, [dlv, feat], vals)
                            return 0

                        lax.fori_loop(0, _D // 16, fgroup, 0)
                return 0

            lax.fori_loop(b * _BLK, (b + 1) * _BLK, chunk, 0)
        return 0

    lax.fori_loop(0, _MAXCH // _BLK, block, 0)

    pltpu.sync_copy(acc.at[pl.ds(0, _RT)],
                    out_hbm.at[pl.ds(pl.multiple_of(w * _RT, _RT), _RT)])


def _norm_from(dp_ref):
    deg = jnp.sum(dp_ref[...], axis=1, keepdims=True)
    return lax.rsqrt(jnp.maximum(deg, 1.0))


def _pre_body(dp_ref, x_ref, w1_ref, g1_ref):
    norm = _norm_from(dp_ref)
    g1_ref[...] = lax.dot_general(
        x_ref[...] * norm, w1_ref[...], (((1,), (0,)), ((), ())),
        precision=lax.Precision.HIGHEST, preferred_element_type=jnp.float32)


def _mid_body(p_ref, dp_ref, b1_ref, w2_ref, g2_ref):
    norm = _norm_from(dp_ref)
    h = jnp.maximum(p_ref[...] * norm + b1_ref[...], 0.0)
    g2_ref[...] = lax.dot_general(
        h * norm, w2_ref[...], (((1,), (0,)), ((), ())),
        precision=lax.Precision.HIGHEST, preferred_element_type=jnp.float32)


def _post_body(q_ref, dp_ref, b2_ref, out_ref):
    norm = _norm_from(dp_ref)
    out_ref[...] = q_ref[...] * norm + b2_ref[...]


_BR = 1000


def _row_spec(cols):
    return pl.BlockSpec((_BR, cols), lambda i: (i, 0))


def _full_spec(r, cols):
    return pl.BlockSpec((r, cols), lambda i: (0, 0))


_pre_call = pl.pallas_call(
    _pre_body,
    grid=(_N // _BR,),
    in_specs=[_row_spec(16), _row_spec(_D), _full_spec(_D, _D)],
    out_specs=_row_spec(_D),
    out_shape=jax.ShapeDtypeStruct((_N, _D), jnp.float32),
)

_mid_call = pl.pallas_call(
    _mid_body,
    grid=(_N // _BR,),
    in_specs=[_row_spec(_D), _row_spec(16), _full_spec(1, _D),
              _full_spec(_D, _D)],
    out_specs=_row_spec(_D),
    out_shape=jax.ShapeDtypeStruct((_N, _D), jnp.float32),
)

_post_call = pl.pallas_call(
    _post_body,
    grid=(_N // _BR,),
    in_specs=[_row_spec(_D), _row_spec(16), _full_spec(1, _D)],
    out_specs=_row_spec(_D),
    out_shape=jax.ShapeDtypeStruct((_N, _D), jnp.float32),
)


@jax.jit
def kernel(features, edge_index, W1, b1, W2, b2):
    src = edge_index[0]
    dst = edge_index[1]
    pad = _EPAD - _E
    src_p = jnp.concatenate([src, jnp.zeros((pad,), jnp.int32)])
    dst_p = jnp.concatenate([dst, jnp.full((pad,), _DUMP, jnp.int32)])

    bsrc, bdst, cnt, deg = _bin_call(src_p, dst_p)
    dp = deg[:_N]

    g1 = _pre_call(dp, features, W1)
    p = _prop_call(g1, bsrc, bdst, cnt)
    g2 = _mid_call(p[:_N], dp, b1.reshape(1, _D), W2)
    q = _prop_call(g2, bsrc, bdst, cnt)
    out = _post_call(q[:_N], dp, b2.reshape(1, _D))
    return out
